# baseline (speedup 1.0000x reference)
import functools

import jax
import jax.numpy as jnp
from jax.experimental import pallas as pl
from jax.experimental.pallas import tpu as pltpu

N = 200
B = 16
D_MODEL = 128
HEADS = 4
NUM_CLASSES = 2
_F32 = jnp.float32
_HI = jax.lax.Precision.HIGHEST


def _leaky(x):
    return jnp.where(x >= 0, x, 0.2 * x)


def _gat_kernel(sc_ref, sct_ref,
                w0_ref, as0_ref, ad0_ref, b0_ref,
                w1_ref, as1_ref, ad1_ref, b1_ref,
                w2_ref, as2_ref, ad2_ref, b2_ref,
                bnw_ref, bnb_ref,
                out_ref):
    x = sc_ref[0]
    row = jax.lax.broadcasted_iota(jnp.int32, (N, N), 0)
    col = jax.lax.broadcasted_iota(jnp.int32, (N, N), 1)
    adjf = jnp.where((sct_ref[0] != 0.0) | (row == col), 1.0, 0.0)

    layer_cfg = (
        (w0_ref, as0_ref, ad0_ref, b0_ref, HEADS, D_MODEL // HEADS),
        (w1_ref, as1_ref, ad1_ref, b1_ref, HEADS, D_MODEL // HEADS),
        (w2_ref, as2_ref, ad2_ref, b2_ref, 1, D_MODEL),
    )
    for j, (w_ref, asrc_ref, adst_ref, bias_ref, heads, dh) in enumerate(layer_cfg):
        h = jnp.dot(x, w_ref[...], preferred_element_type=_F32, precision=_HI)
        e_src = jnp.dot(h, asrc_ref[...], preferred_element_type=_F32, precision=_HI)
        e_dst = jnp.dot(h, adst_ref[...], preferred_element_type=_F32, precision=_HI)
        e_src_t = e_src.T
        outs = []
        for k in range(heads):
            lg = e_dst[:, k:k + 1] + e_src_t[k:k + 1, :]
            p = adjf * jnp.exp(jnp.minimum(_leaky(lg), 60.0))
            alpha = p / jnp.sum(p, axis=1, keepdims=True)
            outs.append(jnp.dot(alpha, h[:, k * dh:(k + 1) * dh],
                                preferred_element_type=_F32, precision=_HI))
        out = outs[0] if heads == 1 else jnp.concatenate(outs, axis=1)
        out = out + bias_ref[...]
        x = out * bnw_ref[j:j + 1, :] + bnb_ref[j:j + 1, :]
        if j < 2:
            x = jnp.where(x > 0, x, jnp.exp(jnp.minimum(x, 0.0)) - 1.0)

    out_ref[0] = x


_KB = 20
_KBLK = (N * D_MODEL) // _KB


def _mlp_kernel(x_ref, w1_ref, b1_ref, ln1w_ref, ln1b_ref,
                w2_ref, b2_ref, ln2w_ref, ln2b_ref,
                w3_ref, b3_ref, out_ref, acc_ref):
    k = pl.program_id(0)

    @pl.when(k == 0)
    def _():
        acc_ref[...] = jnp.zeros_like(acc_ref)

    acc_ref[...] += jnp.dot(x_ref[...], w1_ref[...], preferred_element_type=_F32, precision=_HI)

    @pl.when(k == _KB - 1)
    def _():
        y = acc_ref[...] + b1_ref[...]
        mu = jnp.mean(y, axis=-1, keepdims=True)
        var = jnp.mean((y - mu) ** 2, axis=-1, keepdims=True)
        y = (y - mu) * jax.lax.rsqrt(var + 1e-5) * ln1w_ref[...] + ln1b_ref[...]
        y = jnp.maximum(y, 0.0)
        y = jnp.dot(y, w2_ref[...], preferred_element_type=_F32, precision=_HI) + b2_ref[...]
        mu = jnp.mean(y, axis=-1, keepdims=True)
        var = jnp.mean((y - mu) ** 2, axis=-1, keepdims=True)
        y = (y - mu) * jax.lax.rsqrt(var + 1e-5) * ln2w_ref[...] + ln2b_ref[...]
        y = jnp.maximum(y, 0.0)
        out_ref[...] = jnp.dot(y, w3_ref[...], preferred_element_type=_F32, precision=_HI) + b3_ref[...]


def _att_mat(att, heads, dh):
    a = att[:, :, None] * jnp.eye(heads, dtype=att.dtype)[:, None, :]
    return a.reshape(heads * dh, heads)


@jax.jit
def kernel(fc_matrix, sc_matrix, params):
    del fc_matrix
    sc_t = jnp.swapaxes(sc_matrix, 1, 2)

    dh = D_MODEL // HEADS
    bn_scale = 1.0 / jnp.sqrt(jnp.float32(1.0 + 1e-5))
    bnw = jnp.stack([params['bn%d_w' % j] * bn_scale for j in range(3)])
    bnb = jnp.stack([params['bn%d_b' % j] for j in range(3)])

    gat_args = [sc_matrix, sc_t]
    gat_specs = [
        pl.BlockSpec((1, N, N), lambda b: (b, 0, 0)),
        pl.BlockSpec((1, N, N), lambda b: (b, 0, 0)),
    ]
    for j, (heads, d) in enumerate(((HEADS, dh), (HEADS, dh), (1, D_MODEL))):
        p = params['conv%d' % j]
        gat_args += [p['W'], _att_mat(p['att_src'], heads, d),
                     _att_mat(p['att_dst'], heads, d), p['bias'].reshape(1, D_MODEL)]
        gat_specs += [pl.BlockSpec(p['W'].shape, lambda b: (0, 0)),
                      pl.BlockSpec((heads * d, heads), lambda b: (0, 0)),
                      pl.BlockSpec((heads * d, heads), lambda b: (0, 0)),
                      pl.BlockSpec((1, D_MODEL), lambda b: (0, 0))]
    gat_args += [bnw, bnb]
    gat_specs += [pl.BlockSpec((3, D_MODEL), lambda b: (0, 0)),
                  pl.BlockSpec((3, D_MODEL), lambda b: (0, 0))]

    gat_out = pl.pallas_call(
        _gat_kernel,
        grid=(B,),
        in_specs=gat_specs,
        out_specs=pl.BlockSpec((1, N, D_MODEL), lambda b: (b, 0, 0)),
        out_shape=jax.ShapeDtypeStruct((B, N, D_MODEL), _F32),
        compiler_params=pltpu.CompilerParams(
            dimension_semantics=(pltpu.PARALLEL,)),
    )(*gat_args)

    x_flat = gat_out.reshape(B, N * D_MODEL)

    mlp_args = [
        x_flat, params['W1'], params['b1'].reshape(1, 256),
        params['ln1_w'].reshape(1, 256), params['ln1_b'].reshape(1, 256),
        params['W2'], params['b2'].reshape(1, 64),
        params['ln2_w'].reshape(1, 64), params['ln2_b'].reshape(1, 64),
        params['W3'], params['b3'].reshape(1, NUM_CLASSES),
    ]
    mlp_specs = [
        pl.BlockSpec((B, _KBLK), lambda k: (0, k)),
        pl.BlockSpec((_KBLK, 256), lambda k: (k, 0)),
        pl.BlockSpec((1, 256), lambda k: (0, 0)),
        pl.BlockSpec((1, 256), lambda k: (0, 0)),
        pl.BlockSpec((1, 256), lambda k: (0, 0)),
        pl.BlockSpec((256, 64), lambda k: (0, 0)),
        pl.BlockSpec((1, 64), lambda k: (0, 0)),
        pl.BlockSpec((1, 64), lambda k: (0, 0)),
        pl.BlockSpec((1, 64), lambda k: (0, 0)),
        pl.BlockSpec((64, NUM_CLASSES), lambda k: (0, 0)),
        pl.BlockSpec((1, NUM_CLASSES), lambda k: (0, 0)),
    ]
    out = pl.pallas_call(
        _mlp_kernel,
        grid=(_KB,),
        in_specs=mlp_specs,
        out_specs=pl.BlockSpec((B, NUM_CLASSES), lambda k: (0, 0)),
        out_shape=jax.ShapeDtypeStruct((B, NUM_CLASSES), _F32),
        scratch_shapes=[pltpu.VMEM((B, 256), _F32)],
        compiler_params=pltpu.CompilerParams(
            dimension_semantics=(pltpu.ARBITRARY,)),
    )(*mlp_args)
    return out


# speedup vs baseline: 1.3417x; 1.3417x over previous
import functools

import jax
import jax.numpy as jnp
from jax.experimental import pallas as pl
from jax.experimental.pallas import tpu as pltpu

N = 200
B = 16
D_MODEL = 128
HEADS = 4
NUM_CLASSES = 2
_F32 = jnp.float32
_HI = jax.lax.Precision.HIGHEST


def _leaky(x):
    return jnp.where(x >= 0, x, 0.2 * x)


def _gat_kernel(sc_ref, sct_ref,
                w0_ref, as0_ref, ad0_ref, b0_ref,
                w1_ref, as1_ref, ad1_ref, b1_ref,
                w2_ref, as2_ref, ad2_ref, b2_ref,
                bnw_ref, bnb_ref,
                out_ref):
    x = sc_ref[0]
    row = jax.lax.broadcasted_iota(jnp.int32, (N, N), 0)
    col = jax.lax.broadcasted_iota(jnp.int32, (N, N), 1)
    adjf = jnp.where((sct_ref[0] != 0.0) | (row == col), 1.0, 0.0)

    layer_cfg = (
        (w0_ref, as0_ref, ad0_ref, b0_ref, HEADS, D_MODEL // HEADS),
        (w1_ref, as1_ref, ad1_ref, b1_ref, HEADS, D_MODEL // HEADS),
        (w2_ref, as2_ref, ad2_ref, b2_ref, 1, D_MODEL),
    )
    for j, (w_ref, asrc_ref, adst_ref, bias_ref, heads, dh) in enumerate(layer_cfg):
        h = jnp.dot(x, w_ref[...], preferred_element_type=_F32)
        e_src = jnp.dot(h, asrc_ref[...], preferred_element_type=_F32, precision=_HI)
        e_dst = jnp.dot(h, adst_ref[...], preferred_element_type=_F32, precision=_HI)
        e_src_t = e_src.T
        outs = []
        for k in range(heads):
            lg = e_dst[:, k:k + 1] + e_src_t[k:k + 1, :]
            p = adjf * jnp.exp(jnp.minimum(_leaky(lg), 60.0))
            alpha = p / jnp.sum(p, axis=1, keepdims=True)
            outs.append(jnp.dot(alpha, h[:, k * dh:(k + 1) * dh],
                                preferred_element_type=_F32))
        out = outs[0] if heads == 1 else jnp.concatenate(outs, axis=1)
        out = out + bias_ref[...]
        x = out * bnw_ref[j:j + 1, :] + bnb_ref[j:j + 1, :]
        if j < 2:
            x = jnp.where(x > 0, x, jnp.exp(jnp.minimum(x, 0.0)) - 1.0)

    out_ref[0] = x


_KB = 20
_KBLK = (N * D_MODEL) // _KB


def _mlp_kernel(x_ref, w1_ref, b1_ref, ln1w_ref, ln1b_ref,
                w2_ref, b2_ref, ln2w_ref, ln2b_ref,
                w3_ref, b3_ref, out_ref, acc_ref):
    k = pl.program_id(0)

    @pl.when(k == 0)
    def _():
        acc_ref[...] = jnp.zeros_like(acc_ref)

    acc_ref[...] += jnp.dot(x_ref[...], w1_ref[...], preferred_element_type=_F32)

    @pl.when(k == _KB - 1)
    def _():
        y = acc_ref[...] + b1_ref[...]
        mu = jnp.mean(y, axis=-1, keepdims=True)
        var = jnp.mean((y - mu) ** 2, axis=-1, keepdims=True)
        y = (y - mu) * jax.lax.rsqrt(var + 1e-5) * ln1w_ref[...] + ln1b_ref[...]
        y = jnp.maximum(y, 0.0)
        y = jnp.dot(y, w2_ref[...], preferred_element_type=_F32) + b2_ref[...]
        mu = jnp.mean(y, axis=-1, keepdims=True)
        var = jnp.mean((y - mu) ** 2, axis=-1, keepdims=True)
        y = (y - mu) * jax.lax.rsqrt(var + 1e-5) * ln2w_ref[...] + ln2b_ref[...]
        y = jnp.maximum(y, 0.0)
        out_ref[...] = jnp.dot(y, w3_ref[...], preferred_element_type=_F32) + b3_ref[...]


def _att_mat(att, heads, dh):
    a = att[:, :, None] * jnp.eye(heads, dtype=att.dtype)[:, None, :]
    return a.reshape(heads * dh, heads)


@jax.jit
def kernel(fc_matrix, sc_matrix, params):
    del fc_matrix
    sc_t = jnp.swapaxes(sc_matrix, 1, 2)

    dh = D_MODEL // HEADS
    bn_scale = 1.0 / jnp.sqrt(jnp.float32(1.0 + 1e-5))
    bnw = jnp.stack([params['bn%d_w' % j] * bn_scale for j in range(3)])
    bnb = jnp.stack([params['bn%d_b' % j] for j in range(3)])

    gat_args = [sc_matrix, sc_t]
    gat_specs = [
        pl.BlockSpec((1, N, N), lambda b: (b, 0, 0)),
        pl.BlockSpec((1, N, N), lambda b: (b, 0, 0)),
    ]
    for j, (heads, d) in enumerate(((HEADS, dh), (HEADS, dh), (1, D_MODEL))):
        p = params['conv%d' % j]
        gat_args += [p['W'], _att_mat(p['att_src'], heads, d),
                     _att_mat(p['att_dst'], heads, d), p['bias'].reshape(1, D_MODEL)]
        gat_specs += [pl.BlockSpec(p['W'].shape, lambda b: (0, 0)),
                      pl.BlockSpec((heads * d, heads), lambda b: (0, 0)),
                      pl.BlockSpec((heads * d, heads), lambda b: (0, 0)),
                      pl.BlockSpec((1, D_MODEL), lambda b: (0, 0))]
    gat_args += [bnw, bnb]
    gat_specs += [pl.BlockSpec((3, D_MODEL), lambda b: (0, 0)),
                  pl.BlockSpec((3, D_MODEL), lambda b: (0, 0))]

    gat_out = pl.pallas_call(
        _gat_kernel,
        grid=(B,),
        in_specs=gat_specs,
        out_specs=pl.BlockSpec((1, N, D_MODEL), lambda b: (b, 0, 0)),
        out_shape=jax.ShapeDtypeStruct((B, N, D_MODEL), _F32),
        compiler_params=pltpu.CompilerParams(
            dimension_semantics=(pltpu.PARALLEL,)),
    )(*gat_args)

    x_flat = gat_out.reshape(B, N * D_MODEL)

    mlp_args = [
        x_flat, params['W1'], params['b1'].reshape(1, 256),
        params['ln1_w'].reshape(1, 256), params['ln1_b'].reshape(1, 256),
        params['W2'], params['b2'].reshape(1, 64),
        params['ln2_w'].reshape(1, 64), params['ln2_b'].reshape(1, 64),
        params['W3'], params['b3'].reshape(1, NUM_CLASSES),
    ]
    mlp_specs = [
        pl.BlockSpec((B, _KBLK), lambda k: (0, k)),
        pl.BlockSpec((_KBLK, 256), lambda k: (k, 0)),
        pl.BlockSpec((1, 256), lambda k: (0, 0)),
        pl.BlockSpec((1, 256), lambda k: (0, 0)),
        pl.BlockSpec((1, 256), lambda k: (0, 0)),
        pl.BlockSpec((256, 64), lambda k: (0, 0)),
        pl.BlockSpec((1, 64), lambda k: (0, 0)),
        pl.BlockSpec((1, 64), lambda k: (0, 0)),
        pl.BlockSpec((1, 64), lambda k: (0, 0)),
        pl.BlockSpec((64, NUM_CLASSES), lambda k: (0, 0)),
        pl.BlockSpec((1, NUM_CLASSES), lambda k: (0, 0)),
    ]
    out = pl.pallas_call(
        _mlp_kernel,
        grid=(_KB,),
        in_specs=mlp_specs,
        out_specs=pl.BlockSpec((B, NUM_CLASSES), lambda k: (0, 0)),
        out_shape=jax.ShapeDtypeStruct((B, NUM_CLASSES), _F32),
        scratch_shapes=[pltpu.VMEM((B, 256), _F32)],
        compiler_params=pltpu.CompilerParams(
            dimension_semantics=(pltpu.ARBITRARY,)),
    )(*mlp_args)
    return out


# speedup vs baseline: 1.3778x; 1.0270x over previous
import functools

import jax
import jax.numpy as jnp
from jax.experimental import pallas as pl
from jax.experimental.pallas import tpu as pltpu

N = 200
B = 16
D_MODEL = 128
HEADS = 4
NUM_CLASSES = 2
_F32 = jnp.float32
_MED = jax.lax.Precision.HIGHEST


def _leaky(x):
    return jnp.where(x >= 0, x, 0.2 * x)


_SPS = 2


def _gat_kernel(sc_ref, sct_ref,
                w0_ref, as0_ref, ad0_ref, b0_ref,
                w1_ref, as1_ref, ad1_ref, b1_ref,
                w2_ref, as2_ref, ad2_ref, b2_ref,
                bnw_ref, bnb_ref,
                out_ref):
    row = jax.lax.broadcasted_iota(jnp.int32, (N, N), 0)
    col = jax.lax.broadcasted_iota(jnp.int32, (N, N), 1)
    eye = row == col
    layer_cfg = (
        (w0_ref, as0_ref, ad0_ref, b0_ref, HEADS, D_MODEL // HEADS),
        (w1_ref, as1_ref, ad1_ref, b1_ref, HEADS, D_MODEL // HEADS),
        (w2_ref, as2_ref, ad2_ref, b2_ref, 1, D_MODEL),
    )
    for s in range(_SPS):
        x = sc_ref[s]
        adjf = jnp.where((sct_ref[s] != 0.0) | eye, 1.0, 0.0)
        for j, (w_ref, asrc_ref, adst_ref, bias_ref, heads, dh) in enumerate(layer_cfg):
            h = jnp.dot(x, w_ref[...], preferred_element_type=_F32)
            e_src = jnp.dot(h, asrc_ref[...], preferred_element_type=_F32, precision=_MED)
            e_dst = jnp.dot(h, adst_ref[...], preferred_element_type=_F32, precision=_MED)
            e_src_t = e_src.T
            outs = []
            for k in range(heads):
                lg = e_dst[:, k:k + 1] + e_src_t[k:k + 1, :]
                p = adjf * jnp.exp(jnp.minimum(_leaky(lg), 60.0))
                alpha = p / jnp.sum(p, axis=1, keepdims=True)
                outs.append(jnp.dot(alpha, h[:, k * dh:(k + 1) * dh],
                                    preferred_element_type=_F32))
            out = outs[0] if heads == 1 else jnp.concatenate(outs, axis=1)
            out = out + bias_ref[...]
            x = out * bnw_ref[j:j + 1, :] + bnb_ref[j:j + 1, :]
            if j < 2:
                x = jnp.where(x > 0, x, jnp.exp(jnp.minimum(x, 0.0)) - 1.0)

        out_ref[s] = x


_KB = 20
_KBLK = (N * D_MODEL) // _KB


def _mlp_kernel(x_ref, w1_ref, b1_ref, ln1w_ref, ln1b_ref,
                w2_ref, b2_ref, ln2w_ref, ln2b_ref,
                w3_ref, b3_ref, out_ref, acc_ref):
    k = pl.program_id(0)

    @pl.when(k == 0)
    def _():
        acc_ref[...] = jnp.zeros_like(acc_ref)

    acc_ref[...] += jnp.dot(x_ref[...], w1_ref[...], preferred_element_type=_F32)

    @pl.when(k == _KB - 1)
    def _():
        y = acc_ref[...] + b1_ref[...]
        mu = jnp.mean(y, axis=-1, keepdims=True)
        var = jnp.mean((y - mu) ** 2, axis=-1, keepdims=True)
        y = (y - mu) * jax.lax.rsqrt(var + 1e-5) * ln1w_ref[...] + ln1b_ref[...]
        y = jnp.maximum(y, 0.0)
        y = jnp.dot(y, w2_ref[...], preferred_element_type=_F32) + b2_ref[...]
        mu = jnp.mean(y, axis=-1, keepdims=True)
        var = jnp.mean((y - mu) ** 2, axis=-1, keepdims=True)
        y = (y - mu) * jax.lax.rsqrt(var + 1e-5) * ln2w_ref[...] + ln2b_ref[...]
        y = jnp.maximum(y, 0.0)
        out_ref[...] = jnp.dot(y, w3_ref[...], preferred_element_type=_F32) + b3_ref[...]


def _att_mat(att, heads, dh):
    a = att[:, :, None] * jnp.eye(heads, dtype=att.dtype)[:, None, :]
    return a.reshape(heads * dh, heads)


@jax.jit
def kernel(fc_matrix, sc_matrix, params):
    del fc_matrix
    sc_t = jnp.swapaxes(sc_matrix, 1, 2)

    dh = D_MODEL // HEADS
    bn_scale = 1.0 / jnp.sqrt(jnp.float32(1.0 + 1e-5))
    bnw = jnp.stack([params['bn%d_w' % j] * bn_scale for j in range(3)])
    bnb = jnp.stack([params['bn%d_b' % j] for j in range(3)])

    gat_args = [sc_matrix, sc_t]
    gat_specs = [
        pl.BlockSpec((_SPS, N, N), lambda b: (b, 0, 0)),
        pl.BlockSpec((_SPS, N, N), lambda b: (b, 0, 0)),
    ]
    for j, (heads, d) in enumerate(((HEADS, dh), (HEADS, dh), (1, D_MODEL))):
        p = params['conv%d' % j]
        gat_args += [p['W'], _att_mat(p['att_src'], heads, d),
                     _att_mat(p['att_dst'], heads, d), p['bias'].reshape(1, D_MODEL)]
        gat_specs += [pl.BlockSpec(p['W'].shape, lambda b: (0, 0)),
                      pl.BlockSpec((heads * d, heads), lambda b: (0, 0)),
                      pl.BlockSpec((heads * d, heads), lambda b: (0, 0)),
                      pl.BlockSpec((1, D_MODEL), lambda b: (0, 0))]
    gat_args += [bnw, bnb]
    gat_specs += [pl.BlockSpec((3, D_MODEL), lambda b: (0, 0)),
                  pl.BlockSpec((3, D_MODEL), lambda b: (0, 0))]

    gat_out = pl.pallas_call(
        _gat_kernel,
        grid=(B // _SPS,),
        in_specs=gat_specs,
        out_specs=pl.BlockSpec((_SPS, N, D_MODEL), lambda b: (b, 0, 0)),
        out_shape=jax.ShapeDtypeStruct((B, N, D_MODEL), _F32),
        compiler_params=pltpu.CompilerParams(
            dimension_semantics=(pltpu.PARALLEL,)),
    )(*gat_args)

    x_flat = gat_out.reshape(B, N * D_MODEL)

    mlp_args = [
        x_flat, params['W1'], params['b1'].reshape(1, 256),
        params['ln1_w'].reshape(1, 256), params['ln1_b'].reshape(1, 256),
        params['W2'], params['b2'].reshape(1, 64),
        params['ln2_w'].reshape(1, 64), params['ln2_b'].reshape(1, 64),
        params['W3'], params['b3'].reshape(1, NUM_CLASSES),
    ]
    mlp_specs = [
        pl.BlockSpec((B, _KBLK), lambda k: (0, k)),
        pl.BlockSpec((_KBLK, 256), lambda k: (k, 0)),
        pl.BlockSpec((1, 256), lambda k: (0, 0)),
        pl.BlockSpec((1, 256), lambda k: (0, 0)),
        pl.BlockSpec((1, 256), lambda k: (0, 0)),
        pl.BlockSpec((256, 64), lambda k: (0, 0)),
        pl.BlockSpec((1, 64), lambda k: (0, 0)),
        pl.BlockSpec((1, 64), lambda k: (0, 0)),
        pl.BlockSpec((1, 64), lambda k: (0, 0)),
        pl.BlockSpec((64, NUM_CLASSES), lambda k: (0, 0)),
        pl.BlockSpec((1, NUM_CLASSES), lambda k: (0, 0)),
    ]
    out = pl.pallas_call(
        _mlp_kernel,
        grid=(_KB,),
        in_specs=mlp_specs,
        out_specs=pl.BlockSpec((B, NUM_CLASSES), lambda k: (0, 0)),
        out_shape=jax.ShapeDtypeStruct((B, NUM_CLASSES), _F32),
        scratch_shapes=[pltpu.VMEM((B, 256), _F32)],
        compiler_params=pltpu.CompilerParams(
            dimension_semantics=(pltpu.ARBITRARY,)),
    )(*mlp_args)
    return out


# speedup vs baseline: 1.4299x; 1.0378x over previous
import functools

import jax
import jax.numpy as jnp
from jax.experimental import pallas as pl
from jax.experimental.pallas import tpu as pltpu

N = 200
B = 16
D_MODEL = 128
HEADS = 4
NUM_CLASSES = 2
_F32 = jnp.float32
_MED = jax.lax.Precision.HIGHEST


def _leaky(x):
    return jnp.where(x >= 0, x, 0.2 * x)


_SPS = 2


def _gat_kernel(sc_ref, sct_ref,
                w0_ref, as0_ref, ad0_ref,
                w1_ref, as1_ref, ad1_ref,
                w2_ref, as2_ref, ad2_ref,
                bnw_ref, bnb_ref,
                out_ref):
    row = jax.lax.broadcasted_iota(jnp.int32, (N, N), 0)
    col = jax.lax.broadcasted_iota(jnp.int32, (N, N), 1)
    eye = row == col
    layer_cfg = (
        (w0_ref, as0_ref, ad0_ref, HEADS, D_MODEL // HEADS),
        (w1_ref, as1_ref, ad1_ref, HEADS, D_MODEL // HEADS),
        (w2_ref, as2_ref, ad2_ref, 1, D_MODEL),
    )
    for s in range(_SPS):
        x = sc_ref[s]
        adjf = jnp.where((sct_ref[s] != 0.0) | eye, 1.0, 0.0)
        for j, (w_ref, asrc_ref, adst_ref, heads, dh) in enumerate(layer_cfg):
            h = jnp.dot(x, w_ref[...], preferred_element_type=_F32)
            t = h * asrc_ref[...]
            u = h * adst_ref[...]
            e_src_cols = [jnp.sum(t[:, k * dh:(k + 1) * dh], axis=1, keepdims=True)
                          for k in range(heads)]
            e_dst_cols = [jnp.sum(u[:, k * dh:(k + 1) * dh], axis=1, keepdims=True)
                          for k in range(heads)]
            e_src = e_src_cols[0] if heads == 1 else jnp.concatenate(e_src_cols, axis=1)
            e_src_t = e_src.T
            outs = []
            for k in range(heads):
                lg = e_dst_cols[k] + e_src_t[k:k + 1, :]
                p = adjf * jnp.exp(jnp.minimum(jnp.maximum(lg, 0.2 * lg), 60.0))
                alpha = p / jnp.sum(p, axis=1, keepdims=True)
                outs.append(jnp.dot(alpha, h[:, k * dh:(k + 1) * dh],
                                    preferred_element_type=_F32))
            out = outs[0] if heads == 1 else jnp.concatenate(outs, axis=1)
            x = out * bnw_ref[j:j + 1, :] + bnb_ref[j:j + 1, :]
            if j < 2:
                x = jnp.where(x > 0, x, jnp.exp(jnp.minimum(x, 0.0)) - 1.0)

        out_ref[s] = x


_KB = 20
_KBLK = (N * D_MODEL) // _KB


def _mlp_kernel(x_ref, w1_ref, b1_ref, ln1w_ref, ln1b_ref,
                w2_ref, b2_ref, ln2w_ref, ln2b_ref,
                w3_ref, b3_ref, out_ref, acc_ref):
    k = pl.program_id(0)

    @pl.when(k == 0)
    def _():
        acc_ref[...] = jnp.zeros_like(acc_ref)

    acc_ref[...] += jnp.dot(x_ref[...], w1_ref[...], preferred_element_type=_F32)

    @pl.when(k == _KB - 1)
    def _():
        y = acc_ref[...] + b1_ref[...]
        mu = jnp.mean(y, axis=-1, keepdims=True)
        var = jnp.mean((y - mu) ** 2, axis=-1, keepdims=True)
        y = (y - mu) * jax.lax.rsqrt(var + 1e-5) * ln1w_ref[...] + ln1b_ref[...]
        y = jnp.maximum(y, 0.0)
        y = jnp.dot(y, w2_ref[...], preferred_element_type=_F32) + b2_ref[...]
        mu = jnp.mean(y, axis=-1, keepdims=True)
        var = jnp.mean((y - mu) ** 2, axis=-1, keepdims=True)
        y = (y - mu) * jax.lax.rsqrt(var + 1e-5) * ln2w_ref[...] + ln2b_ref[...]
        y = jnp.maximum(y, 0.0)
        out_ref[...] = jnp.dot(y, w3_ref[...], preferred_element_type=_F32) + b3_ref[...]


def _att_mat(att, heads, dh):
    a = att[:, :, None] * jnp.eye(heads, dtype=att.dtype)[:, None, :]
    return a.reshape(heads * dh, heads)


@jax.jit
def kernel(fc_matrix, sc_matrix, params):
    del fc_matrix
    sc_t = jnp.swapaxes(sc_matrix, 1, 2)

    dh = D_MODEL // HEADS
    bn_scale = 1.0 / jnp.sqrt(jnp.float32(1.0 + 1e-5))
    bnw_l = [params['bn%d_w' % j] * bn_scale for j in range(3)]
    bnw = jnp.stack(bnw_l)
    bnb = jnp.stack([params['conv%d' % j]['bias'] * bnw_l[j]
                     + params['bn%d_b' % j] for j in range(3)])

    gat_args = [sc_matrix, sc_t]
    gat_specs = [
        pl.BlockSpec((_SPS, N, N), lambda b: (b, 0, 0)),
        pl.BlockSpec((_SPS, N, N), lambda b: (b, 0, 0)),
    ]
    for j, (heads, d) in enumerate(((HEADS, dh), (HEADS, dh), (1, D_MODEL))):
        p = params['conv%d' % j]
        gat_args += [p['W'], p['att_src'].reshape(1, D_MODEL),
                     p['att_dst'].reshape(1, D_MODEL)]
        gat_specs += [pl.BlockSpec(p['W'].shape, lambda b: (0, 0)),
                      pl.BlockSpec((1, D_MODEL), lambda b: (0, 0)),
                      pl.BlockSpec((1, D_MODEL), lambda b: (0, 0))]
    gat_args += [bnw, bnb]
    gat_specs += [pl.BlockSpec((3, D_MODEL), lambda b: (0, 0)),
                  pl.BlockSpec((3, D_MODEL), lambda b: (0, 0))]

    gat_out = pl.pallas_call(
        _gat_kernel,
        grid=(B // _SPS,),
        in_specs=gat_specs,
        out_specs=pl.BlockSpec((_SPS, N, D_MODEL), lambda b: (b, 0, 0)),
        out_shape=jax.ShapeDtypeStruct((B, N, D_MODEL), _F32),
        compiler_params=pltpu.CompilerParams(
            dimension_semantics=(pltpu.PARALLEL,)),
    )(*gat_args)

    x_flat = gat_out.reshape(B, N * D_MODEL)

    mlp_args = [
        x_flat, params['W1'], params['b1'].reshape(1, 256),
        params['ln1_w'].reshape(1, 256), params['ln1_b'].reshape(1, 256),
        params['W2'], params['b2'].reshape(1, 64),
        params['ln2_w'].reshape(1, 64), params['ln2_b'].reshape(1, 64),
        params['W3'], params['b3'].reshape(1, NUM_CLASSES),
    ]
    mlp_specs = [
        pl.BlockSpec((B, _KBLK), lambda k: (0, k)),
        pl.BlockSpec((_KBLK, 256), lambda k: (k, 0)),
        pl.BlockSpec((1, 256), lambda k: (0, 0)),
        pl.BlockSpec((1, 256), lambda k: (0, 0)),
        pl.BlockSpec((1, 256), lambda k: (0, 0)),
        pl.BlockSpec((256, 64), lambda k: (0, 0)),
        pl.BlockSpec((1, 64), lambda k: (0, 0)),
        pl.BlockSpec((1, 64), lambda k: (0, 0)),
        pl.BlockSpec((1, 64), lambda k: (0, 0)),
        pl.BlockSpec((64, NUM_CLASSES), lambda k: (0, 0)),
        pl.BlockSpec((1, NUM_CLASSES), lambda k: (0, 0)),
    ]
    out = pl.pallas_call(
        _mlp_kernel,
        grid=(_KB,),
        in_specs=mlp_specs,
        out_specs=pl.BlockSpec((B, NUM_CLASSES), lambda k: (0, 0)),
        out_shape=jax.ShapeDtypeStruct((B, NUM_CLASSES), _F32),
        scratch_shapes=[pltpu.VMEM((B, 256), _F32)],
        compiler_params=pltpu.CompilerParams(
            dimension_semantics=(pltpu.ARBITRARY,)),
    )(*mlp_args)
    return out


# speedup vs baseline: 1.6531x; 1.1561x over previous
import functools

import jax
import jax.numpy as jnp
from jax.experimental import pallas as pl
from jax.experimental.pallas import tpu as pltpu

N = 200
B = 16
D_MODEL = 128
HEADS = 4
NUM_CLASSES = 2
_F32 = jnp.float32
_MED = jax.lax.Precision.HIGHEST


def _leaky(x):
    return jnp.where(x >= 0, x, 0.2 * x)


_SPS = 2


def _gat_kernel(sc_ref,
                w0_ref, as0_ref, ad0_ref,
                w1_ref, as1_ref, ad1_ref,
                w2_ref, as2_ref, ad2_ref,
                bnw_ref, bnb_ref,
                out_ref):
    row = jax.lax.broadcasted_iota(jnp.int32, (N, N), 0)
    col = jax.lax.broadcasted_iota(jnp.int32, (N, N), 1)
    eye = row == col
    layer_cfg = (
        (w0_ref, as0_ref, ad0_ref, HEADS, D_MODEL // HEADS),
        (w1_ref, as1_ref, ad1_ref, HEADS, D_MODEL // HEADS),
        (w2_ref, as2_ref, ad2_ref, 1, D_MODEL),
    )
    for s in range(_SPS):
        x = sc_ref[s]
        adjf = jnp.where((x != 0.0) | eye, 1.0, 0.0)
        for j, (w_ref, asrc_ref, adst_ref, heads, dh) in enumerate(layer_cfg):
            h = jnp.dot(x, w_ref[...], preferred_element_type=_F32)
            t = h * asrc_ref[...]
            u = h * adst_ref[...]
            e_src_cols = [jnp.sum(t[:, k * dh:(k + 1) * dh], axis=1, keepdims=True)
                          for k in range(heads)]
            e_dst_cols = [jnp.sum(u[:, k * dh:(k + 1) * dh], axis=1, keepdims=True)
                          for k in range(heads)]
            e_dst = e_dst_cols[0] if heads == 1 else jnp.concatenate(e_dst_cols, axis=1)
            e_dst_t = e_dst.T
            outs = []
            for k in range(heads):
                lg = e_src_cols[k] + e_dst_t[k:k + 1, :]
                p = adjf * jnp.exp(jnp.minimum(jnp.maximum(lg, 0.2 * lg), 60.0))
                beta = p / jnp.sum(p, axis=0, keepdims=True)
                outs.append(jax.lax.dot_general(
                    beta, h[:, k * dh:(k + 1) * dh],
                    dimension_numbers=(((0,), (0,)), ((), ())),
                    preferred_element_type=_F32))
            out = outs[0] if heads == 1 else jnp.concatenate(outs, axis=1)
            x = out * bnw_ref[j:j + 1, :] + bnb_ref[j:j + 1, :]
            if j < 2:
                x = jnp.where(x > 0, x, jnp.exp(jnp.minimum(x, 0.0)) - 1.0)

        out_ref[s] = x


_KB = 20
_KBLK = (N * D_MODEL) // _KB


def _mlp_kernel(x_ref, w1_ref, b1_ref, ln1w_ref, ln1b_ref,
                w2_ref, b2_ref, ln2w_ref, ln2b_ref,
                w3_ref, b3_ref, out_ref, acc_ref):
    k = pl.program_id(0)

    @pl.when(k == 0)
    def _():
        acc_ref[...] = jnp.zeros_like(acc_ref)

    acc_ref[...] += jnp.dot(x_ref[...], w1_ref[...], preferred_element_type=_F32)

    @pl.when(k == _KB - 1)
    def _():
        y = acc_ref[...] + b1_ref[...]
        mu = jnp.mean(y, axis=-1, keepdims=True)
        var = jnp.mean((y - mu) ** 2, axis=-1, keepdims=True)
        y = (y - mu) * jax.lax.rsqrt(var + 1e-5) * ln1w_ref[...] + ln1b_ref[...]
        y = jnp.maximum(y, 0.0)
        y = jnp.dot(y, w2_ref[...], preferred_element_type=_F32) + b2_ref[...]
        mu = jnp.mean(y, axis=-1, keepdims=True)
        var = jnp.mean((y - mu) ** 2, axis=-1, keepdims=True)
        y = (y - mu) * jax.lax.rsqrt(var + 1e-5) * ln2w_ref[...] + ln2b_ref[...]
        y = jnp.maximum(y, 0.0)
        out_ref[...] = jnp.dot(y, w3_ref[...], preferred_element_type=_F32) + b3_ref[...]


def _att_mat(att, heads, dh):
    a = att[:, :, None] * jnp.eye(heads, dtype=att.dtype)[:, None, :]
    return a.reshape(heads * dh, heads)


@jax.jit
def kernel(fc_matrix, sc_matrix, params):
    del fc_matrix

    dh = D_MODEL // HEADS
    bn_scale = 1.0 / jnp.sqrt(jnp.float32(1.0 + 1e-5))
    bnw_l = [params['bn%d_w' % j] * bn_scale for j in range(3)]
    bnw = jnp.stack(bnw_l)
    bnb = jnp.stack([params['conv%d' % j]['bias'] * bnw_l[j]
                     + params['bn%d_b' % j] for j in range(3)])

    gat_args = [sc_matrix]
    gat_specs = [
        pl.BlockSpec((_SPS, N, N), lambda b: (b, 0, 0)),
    ]
    for j, (heads, d) in enumerate(((HEADS, dh), (HEADS, dh), (1, D_MODEL))):
        p = params['conv%d' % j]
        gat_args += [p['W'], p['att_src'].reshape(1, D_MODEL),
                     p['att_dst'].reshape(1, D_MODEL)]
        gat_specs += [pl.BlockSpec(p['W'].shape, lambda b: (0, 0)),
                      pl.BlockSpec((1, D_MODEL), lambda b: (0, 0)),
                      pl.BlockSpec((1, D_MODEL), lambda b: (0, 0))]
    gat_args += [bnw, bnb]
    gat_specs += [pl.BlockSpec((3, D_MODEL), lambda b: (0, 0)),
                  pl.BlockSpec((3, D_MODEL), lambda b: (0, 0))]

    gat_out = pl.pallas_call(
        _gat_kernel,
        grid=(B // _SPS,),
        in_specs=gat_specs,
        out_specs=pl.BlockSpec((_SPS, N, D_MODEL), lambda b: (b, 0, 0)),
        out_shape=jax.ShapeDtypeStruct((B, N, D_MODEL), _F32),
        compiler_params=pltpu.CompilerParams(
            dimension_semantics=(pltpu.PARALLEL,),
            fuse_transposed_lhs_in_matmul=True),
    )(*gat_args)

    x_flat = gat_out.reshape(B, N * D_MODEL)

    mlp_args = [
        x_flat, params['W1'], params['b1'].reshape(1, 256),
        params['ln1_w'].reshape(1, 256), params['ln1_b'].reshape(1, 256),
        params['W2'], params['b2'].reshape(1, 64),
        params['ln2_w'].reshape(1, 64), params['ln2_b'].reshape(1, 64),
        params['W3'], params['b3'].reshape(1, NUM_CLASSES),
    ]
    mlp_specs = [
        pl.BlockSpec((B, _KBLK), lambda k: (0, k)),
        pl.BlockSpec((_KBLK, 256), lambda k: (k, 0)),
        pl.BlockSpec((1, 256), lambda k: (0, 0)),
        pl.BlockSpec((1, 256), lambda k: (0, 0)),
        pl.BlockSpec((1, 256), lambda k: (0, 0)),
        pl.BlockSpec((256, 64), lambda k: (0, 0)),
        pl.BlockSpec((1, 64), lambda k: (0, 0)),
        pl.BlockSpec((1, 64), lambda k: (0, 0)),
        pl.BlockSpec((1, 64), lambda k: (0, 0)),
        pl.BlockSpec((64, NUM_CLASSES), lambda k: (0, 0)),
        pl.BlockSpec((1, NUM_CLASSES), lambda k: (0, 0)),
    ]
    out = pl.pallas_call(
        _mlp_kernel,
        grid=(_KB,),
        in_specs=mlp_specs,
        out_specs=pl.BlockSpec((B, NUM_CLASSES), lambda k: (0, 0)),
        out_shape=jax.ShapeDtypeStruct((B, NUM_CLASSES), _F32),
        scratch_shapes=[pltpu.VMEM((B, 256), _F32)],
        compiler_params=pltpu.CompilerParams(
            dimension_semantics=(pltpu.ARBITRARY,)),
    )(*mlp_args)
    return out


# speedup vs baseline: 2.0726x; 1.2538x over previous
import jax
import jax.numpy as jnp
from jax.experimental import pallas as pl
from jax.experimental.pallas import tpu as pltpu

N = 200
B = 16
D_MODEL = 128
HEADS = 4
NUM_CLASSES = 2
_F32 = jnp.float32

_SPS = 2
_GSTEPS = B // _SPS
_STEPS = _GSTEPS + 1


def _fused_kernel(sc_ref,
                  w0_ref, as0_ref, ad0_ref,
                  w1g_ref, as1_ref, ad1_ref,
                  w2g_ref, as2_ref, ad2_ref,
                  bnw_ref, bnb_ref,
                  w1r_ref,
                  b1_ref, ln1w_ref, ln1b_ref,
                  w2_ref, b2_ref, ln2w_ref, ln2b_ref,
                  w3_ref, b3_ref,
                  out_ref,
                  xall_ref, w1s_ref, sem):
    b = pl.program_id(0)

    w1_copy = pltpu.make_async_copy(w1r_ref, w1s_ref, sem)

    @pl.when(b == 0)
    def _():
        w1_copy.start()

    @pl.when(b < _GSTEPS)
    def _():
        row = jax.lax.broadcasted_iota(jnp.int32, (N, N), 0)
        col = jax.lax.broadcasted_iota(jnp.int32, (N, N), 1)
        eye = row == col
        layer_cfg = (
            (w0_ref, as0_ref, ad0_ref, HEADS, D_MODEL // HEADS),
            (w1g_ref, as1_ref, ad1_ref, HEADS, D_MODEL // HEADS),
            (w2g_ref, as2_ref, ad2_ref, 1, D_MODEL),
        )
        for s in range(_SPS):
            x = sc_ref[s]
            adjf = jnp.where((x != 0.0) | eye, 1.0, 0.0)
            for j, (w_ref, asrc_ref, adst_ref, heads, dh) in enumerate(layer_cfg):
                h = jnp.dot(x, w_ref[...], preferred_element_type=_F32)
                t = h * asrc_ref[...]
                u = h * adst_ref[...]
                e_src_cols = [jnp.sum(t[:, k * dh:(k + 1) * dh], axis=1, keepdims=True)
                              for k in range(heads)]
                e_dst_cols = [jnp.sum(u[:, k * dh:(k + 1) * dh], axis=1, keepdims=True)
                              for k in range(heads)]
                e_dst = (e_dst_cols[0] if heads == 1
                         else jnp.concatenate(e_dst_cols, axis=1))
                e_dst_t = e_dst.T
                outs = []
                for k in range(heads):
                    lg = e_src_cols[k] + e_dst_t[k:k + 1, :]
                    p = adjf * jnp.exp(jnp.minimum(jnp.maximum(lg, 0.2 * lg), 60.0))
                    beta = p / jnp.sum(p, axis=0, keepdims=True)
                    outs.append(jax.lax.dot_general(
                        beta, h[:, k * dh:(k + 1) * dh],
                        dimension_numbers=(((0,), (0,)), ((), ())),
                        preferred_element_type=_F32))
                out = outs[0] if heads == 1 else jnp.concatenate(outs, axis=1)
                x = out * bnw_ref[j:j + 1, :] + bnb_ref[j:j + 1, :]
                if j < 2:
                    x = jnp.where(x > 0, x, jnp.exp(jnp.minimum(x, 0.0)) - 1.0)

            xall_ref[pl.ds(b * _SPS + s, 1)] = x[None]

    @pl.when(b == _GSTEPS)
    def _():
        w1_copy.wait()
        acc = jnp.zeros((B, 256), dtype=_F32)
        for n in range(N):
            acc = acc + jnp.dot(xall_ref[:, n, :], w1s_ref[n],
                                preferred_element_type=_F32)
        y = acc + b1_ref[...]
        mu = jnp.mean(y, axis=-1, keepdims=True)
        var = jnp.mean((y - mu) ** 2, axis=-1, keepdims=True)
        y = (y - mu) * jax.lax.rsqrt(var + 1e-5) * ln1w_ref[...] + ln1b_ref[...]
        y = jnp.maximum(y, 0.0)
        y = jnp.dot(y, w2_ref[...], preferred_element_type=_F32) + b2_ref[...]
        mu = jnp.mean(y, axis=-1, keepdims=True)
        var = jnp.mean((y - mu) ** 2, axis=-1, keepdims=True)
        y = (y - mu) * jax.lax.rsqrt(var + 1e-5) * ln2w_ref[...] + ln2b_ref[...]
        y = jnp.maximum(y, 0.0)
        out_ref[...] = jnp.dot(y, w3_ref[...], preferred_element_type=_F32) + b3_ref[...]


@jax.jit
def kernel(fc_matrix, sc_matrix, params):
    del fc_matrix

    bn_scale = 1.0 / jnp.sqrt(jnp.float32(1.0 + 1e-5))
    bnw_l = [params['bn%d_w' % j] * bn_scale for j in range(3)]
    bnw = jnp.stack(bnw_l)
    bnb = jnp.stack([params['conv%d' % j]['bias'] * bnw_l[j]
                     + params['bn%d_b' % j] for j in range(3)])

    last_g = _GSTEPS - 1
    args = [sc_matrix]
    specs = [
        pl.BlockSpec((_SPS, N, N),
                     lambda b: (jnp.minimum(b, last_g), 0, 0)),
    ]
    for j in range(3):
        p = params['conv%d' % j]
        args += [p['W'], p['att_src'].reshape(1, D_MODEL),
                 p['att_dst'].reshape(1, D_MODEL)]
        specs += [pl.BlockSpec(p['W'].shape, lambda b: (0, 0)),
                  pl.BlockSpec((1, D_MODEL), lambda b: (0, 0)),
                  pl.BlockSpec((1, D_MODEL), lambda b: (0, 0))]
    args += [bnw, bnb]
    specs += [pl.BlockSpec((3, D_MODEL), lambda b: (0, 0)),
              pl.BlockSpec((3, D_MODEL), lambda b: (0, 0))]

    args += [params['W1'].reshape(N, D_MODEL, 256)]
    specs += [pl.BlockSpec(memory_space=pl.ANY)]

    args += [
        params['b1'].reshape(1, 256),
        params['ln1_w'].reshape(1, 256), params['ln1_b'].reshape(1, 256),
        params['W2'], params['b2'].reshape(1, 64),
        params['ln2_w'].reshape(1, 64), params['ln2_b'].reshape(1, 64),
        params['W3'], params['b3'].reshape(1, NUM_CLASSES),
    ]
    specs += [
        pl.BlockSpec((1, 256), lambda b: (0, 0)),
        pl.BlockSpec((1, 256), lambda b: (0, 0)),
        pl.BlockSpec((1, 256), lambda b: (0, 0)),
        pl.BlockSpec((256, 64), lambda b: (0, 0)),
        pl.BlockSpec((1, 64), lambda b: (0, 0)),
        pl.BlockSpec((1, 64), lambda b: (0, 0)),
        pl.BlockSpec((1, 64), lambda b: (0, 0)),
        pl.BlockSpec((64, NUM_CLASSES), lambda b: (0, 0)),
        pl.BlockSpec((1, NUM_CLASSES), lambda b: (0, 0)),
    ]

    out = pl.pallas_call(
        _fused_kernel,
        grid=(_STEPS,),
        in_specs=specs,
        out_specs=pl.BlockSpec((B, NUM_CLASSES), lambda b: (0, 0)),
        out_shape=jax.ShapeDtypeStruct((B, NUM_CLASSES), _F32),
        scratch_shapes=[
            pltpu.VMEM((B, N, D_MODEL), _F32),
            pltpu.VMEM((N, D_MODEL, 256), _F32),
            pltpu.SemaphoreType.DMA,
        ],
        compiler_params=pltpu.CompilerParams(
            dimension_semantics=(pltpu.ARBITRARY,),
            fuse_transposed_lhs_in_matmul=True),
    )(*args)
    return out


# speedup vs baseline: 2.0812x; 1.0042x over previous
import jax
import jax.numpy as jnp
from jax.experimental import pallas as pl
from jax.experimental.pallas import tpu as pltpu

N = 200
B = 16
D_MODEL = 128
HEADS = 4
NUM_CLASSES = 2
_F32 = jnp.float32

_SPS = 4
_GSTEPS = B // _SPS
_STEPS = _GSTEPS + 1


def _fused_kernel(sc_ref,
                  w0_ref, as0_ref, ad0_ref,
                  w1g_ref, as1_ref, ad1_ref,
                  w2g_ref, as2_ref, ad2_ref,
                  bnw_ref, bnb_ref,
                  w1r_ref,
                  b1_ref, ln1w_ref, ln1b_ref,
                  w2_ref, b2_ref, ln2w_ref, ln2b_ref,
                  w3_ref, b3_ref,
                  out_ref,
                  xall_ref, w1s_ref, sem):
    b = pl.program_id(0)

    w1_copy = pltpu.make_async_copy(w1r_ref, w1s_ref, sem)

    @pl.when(b == 0)
    def _():
        w1_copy.start()

    @pl.when(b < _GSTEPS)
    def _():
        row = jax.lax.broadcasted_iota(jnp.int32, (N, N), 0)
        col = jax.lax.broadcasted_iota(jnp.int32, (N, N), 1)
        eye = row == col
        layer_cfg = (
            (w0_ref, as0_ref, ad0_ref, HEADS, D_MODEL // HEADS),
            (w1g_ref, as1_ref, ad1_ref, HEADS, D_MODEL // HEADS),
            (w2g_ref, as2_ref, ad2_ref, 1, D_MODEL),
        )
        for s in range(_SPS):
            x = sc_ref[s]
            adjf = jnp.where((x != 0.0) | eye, 1.0, 0.0)
            for j, (w_ref, asrc_ref, adst_ref, heads, dh) in enumerate(layer_cfg):
                h = jnp.dot(x, w_ref[...], preferred_element_type=_F32)
                t = h * asrc_ref[...]
                u = h * adst_ref[...]
                e_src_cols = [jnp.sum(t[:, k * dh:(k + 1) * dh], axis=1, keepdims=True)
                              for k in range(heads)]
                e_dst_cols = [jnp.sum(u[:, k * dh:(k + 1) * dh], axis=1, keepdims=True)
                              for k in range(heads)]
                e_dst = (e_dst_cols[0] if heads == 1
                         else jnp.concatenate(e_dst_cols, axis=1))
                e_dst_t = e_dst.T
                outs = []
                for k in range(heads):
                    lg = e_src_cols[k] + e_dst_t[k:k + 1, :]
                    p = adjf * jnp.exp(jnp.minimum(jnp.maximum(lg, 0.2 * lg), 60.0))
                    beta = p / jnp.sum(p, axis=0, keepdims=True)
                    outs.append(jax.lax.dot_general(
                        beta, h[:, k * dh:(k + 1) * dh],
                        dimension_numbers=(((0,), (0,)), ((), ())),
                        preferred_element_type=_F32))
                out = outs[0] if heads == 1 else jnp.concatenate(outs, axis=1)
                x = out * bnw_ref[j:j + 1, :] + bnb_ref[j:j + 1, :]
                if j < 2:
                    x = jnp.where(x > 0, x, jnp.exp(jnp.minimum(x, 0.0)) - 1.0)

            xall_ref[pl.ds(b * _SPS + s, 1)] = x[None]

    @pl.when(b == _GSTEPS)
    def _():
        w1_copy.wait()
        acc = jnp.zeros((B, 256), dtype=_F32)
        for n in range(N):
            acc = acc + jnp.dot(xall_ref[:, n, :], w1s_ref[n],
                                preferred_element_type=_F32)
        y = acc + b1_ref[...]
        mu = jnp.mean(y, axis=-1, keepdims=True)
        var = jnp.mean((y - mu) ** 2, axis=-1, keepdims=True)
        y = (y - mu) * jax.lax.rsqrt(var + 1e-5) * ln1w_ref[...] + ln1b_ref[...]
        y = jnp.maximum(y, 0.0)
        y = jnp.dot(y, w2_ref[...], preferred_element_type=_F32) + b2_ref[...]
        mu = jnp.mean(y, axis=-1, keepdims=True)
        var = jnp.mean((y - mu) ** 2, axis=-1, keepdims=True)
        y = (y - mu) * jax.lax.rsqrt(var + 1e-5) * ln2w_ref[...] + ln2b_ref[...]
        y = jnp.maximum(y, 0.0)
        out_ref[...] = jnp.dot(y, w3_ref[...], preferred_element_type=_F32) + b3_ref[...]


@jax.jit
def kernel(fc_matrix, sc_matrix, params):
    del fc_matrix

    bn_scale = 1.0 / jnp.sqrt(jnp.float32(1.0 + 1e-5))
    bnw_l = [params['bn%d_w' % j] * bn_scale for j in range(3)]
    bnw = jnp.stack(bnw_l)
    bnb = jnp.stack([params['conv%d' % j]['bias'] * bnw_l[j]
                     + params['bn%d_b' % j] for j in range(3)])

    last_g = _GSTEPS - 1
    args = [sc_matrix]
    specs = [
        pl.BlockSpec((_SPS, N, N),
                     lambda b: (jnp.minimum(b, last_g), 0, 0)),
    ]
    for j in range(3):
        p = params['conv%d' % j]
        args += [p['W'], p['att_src'].reshape(1, D_MODEL),
                 p['att_dst'].reshape(1, D_MODEL)]
        specs += [pl.BlockSpec(p['W'].shape, lambda b: (0, 0)),
                  pl.BlockSpec((1, D_MODEL), lambda b: (0, 0)),
                  pl.BlockSpec((1, D_MODEL), lambda b: (0, 0))]
    args += [bnw, bnb]
    specs += [pl.BlockSpec((3, D_MODEL), lambda b: (0, 0)),
              pl.BlockSpec((3, D_MODEL), lambda b: (0, 0))]

    args += [params['W1'].reshape(N, D_MODEL, 256)]
    specs += [pl.BlockSpec(memory_space=pl.ANY)]

    args += [
        params['b1'].reshape(1, 256),
        params['ln1_w'].reshape(1, 256), params['ln1_b'].reshape(1, 256),
        params['W2'], params['b2'].reshape(1, 64),
        params['ln2_w'].reshape(1, 64), params['ln2_b'].reshape(1, 64),
        params['W3'], params['b3'].reshape(1, NUM_CLASSES),
    ]
    specs += [
        pl.BlockSpec((1, 256), lambda b: (0, 0)),
        pl.BlockSpec((1, 256), lambda b: (0, 0)),
        pl.BlockSpec((1, 256), lambda b: (0, 0)),
        pl.BlockSpec((256, 64), lambda b: (0, 0)),
        pl.BlockSpec((1, 64), lambda b: (0, 0)),
        pl.BlockSpec((1, 64), lambda b: (0, 0)),
        pl.BlockSpec((1, 64), lambda b: (0, 0)),
        pl.BlockSpec((64, NUM_CLASSES), lambda b: (0, 0)),
        pl.BlockSpec((1, NUM_CLASSES), lambda b: (0, 0)),
    ]

    out = pl.pallas_call(
        _fused_kernel,
        grid=(_STEPS,),
        in_specs=specs,
        out_specs=pl.BlockSpec((B, NUM_CLASSES), lambda b: (0, 0)),
        out_shape=jax.ShapeDtypeStruct((B, NUM_CLASSES), _F32),
        scratch_shapes=[
            pltpu.VMEM((B, N, D_MODEL), _F32),
            pltpu.VMEM((N, D_MODEL, 256), _F32),
            pltpu.SemaphoreType.DMA,
        ],
        compiler_params=pltpu.CompilerParams(
            dimension_semantics=(pltpu.ARBITRARY,),
            fuse_transposed_lhs_in_matmul=True),
    )(*args)
    return out


# speedup vs baseline: 2.3689x; 1.1383x over previous
import jax
import jax.numpy as jnp
from jax.experimental import pallas as pl
from jax.experimental.pallas import tpu as pltpu

N = 200
B = 16
D_MODEL = 128
HEADS = 4
NUM_CLASSES = 2
_F32 = jnp.float32

_SPS = 4
_GSTEPS = B // _SPS
_STEPS = _GSTEPS + 1


def _fused_kernel(sc_ref, o4_ref, o1_ref,
                  w0_ref, as0_ref, ad0_ref,
                  w1g_ref, as1_ref, ad1_ref,
                  w2g_ref, as2_ref, ad2_ref,
                  bnw_ref, bnb_ref,
                  w1r_ref,
                  b1_ref, ln1w_ref, ln1b_ref,
                  w2_ref, b2_ref, ln2w_ref, ln2b_ref,
                  w3_ref, b3_ref,
                  out_ref,
                  xall_ref, w1s_ref, sem):
    b = pl.program_id(0)

    w1_copy = pltpu.make_async_copy(w1r_ref, w1s_ref, sem)

    @pl.when(b == 0)
    def _():
        w1_copy.start()

    @pl.when(b < _GSTEPS)
    def _():
        row = jax.lax.broadcasted_iota(jnp.int32, (N, N), 0)
        col = jax.lax.broadcasted_iota(jnp.int32, (N, N), 1)
        eye = row == col
        layer_cfg = (
            (w0_ref, as0_ref, ad0_ref, HEADS, D_MODEL // HEADS),
            (w1g_ref, as1_ref, ad1_ref, HEADS, D_MODEL // HEADS),
            (w2g_ref, as2_ref, ad2_ref, 1, D_MODEL),
        )
        for s in range(_SPS):
            x = sc_ref[s]
            adjf = jnp.where((x != 0.0) | eye, 1.0, 0.0)
            for j, (w_ref, asrc_ref, adst_ref, heads, dh) in enumerate(layer_cfg):
                h = jnp.dot(x, w_ref[...], preferred_element_type=_F32)
                o_ref = o4_ref if heads == HEADS else o1_ref
                t = h * asrc_ref[...]
                u = h * adst_ref[...]
                th = t.astype(jnp.bfloat16).astype(_F32)
                uh = u.astype(jnp.bfloat16).astype(_F32)
                e_src = (jnp.dot(th, o_ref[...], preferred_element_type=_F32)
                         + jnp.dot(t - th, o_ref[...], preferred_element_type=_F32))
                e_dst = (jnp.dot(uh, o_ref[...], preferred_element_type=_F32)
                         + jnp.dot(u - uh, o_ref[...], preferred_element_type=_F32))
                e_src_cols = [e_src[:, k:k + 1] for k in range(heads)]
                e_dst_t = e_dst.T
                outs = []
                for k in range(heads):
                    lg = e_src_cols[k] + e_dst_t[k:k + 1, :]
                    p = adjf * jnp.exp(jnp.minimum(jnp.maximum(lg, 0.2 * lg), 60.0))
                    beta = p / jnp.sum(p, axis=0, keepdims=True)
                    outs.append(jax.lax.dot_general(
                        beta, h[:, k * dh:(k + 1) * dh],
                        dimension_numbers=(((0,), (0,)), ((), ())),
                        preferred_element_type=_F32))
                out = outs[0] if heads == 1 else jnp.concatenate(outs, axis=1)
                x = out * bnw_ref[j:j + 1, :] + bnb_ref[j:j + 1, :]
                if j < 2:
                    x = jnp.where(x > 0, x, jnp.exp(jnp.minimum(x, 0.0)) - 1.0)

            xall_ref[pl.ds(b * _SPS + s, 1)] = x[None]

    @pl.when(b == _GSTEPS)
    def _():
        w1_copy.wait()
        acc = jnp.zeros((B, 256), dtype=_F32)
        for n in range(N):
            acc = acc + jnp.dot(xall_ref[:, n, :], w1s_ref[n],
                                preferred_element_type=_F32)
        y = acc + b1_ref[...]
        mu = jnp.mean(y, axis=-1, keepdims=True)
        var = jnp.mean((y - mu) ** 2, axis=-1, keepdims=True)
        y = (y - mu) * jax.lax.rsqrt(var + 1e-5) * ln1w_ref[...] + ln1b_ref[...]
        y = jnp.maximum(y, 0.0)
        y = jnp.dot(y, w2_ref[...], preferred_element_type=_F32) + b2_ref[...]
        mu = jnp.mean(y, axis=-1, keepdims=True)
        var = jnp.mean((y - mu) ** 2, axis=-1, keepdims=True)
        y = (y - mu) * jax.lax.rsqrt(var + 1e-5) * ln2w_ref[...] + ln2b_ref[...]
        y = jnp.maximum(y, 0.0)
        out_ref[...] = jnp.dot(y, w3_ref[...], preferred_element_type=_F32) + b3_ref[...]


@jax.jit
def kernel(fc_matrix, sc_matrix, params):
    del fc_matrix

    bn_scale = 1.0 / jnp.sqrt(jnp.float32(1.0 + 1e-5))
    bnw_l = [params['bn%d_w' % j] * bn_scale for j in range(3)]
    bnw = jnp.stack(bnw_l)
    bnb = jnp.stack([params['conv%d' % j]['bias'] * bnw_l[j]
                     + params['bn%d_b' % j] for j in range(3)])

    seg = jnp.repeat(jnp.eye(HEADS, dtype=_F32), D_MODEL // HEADS, axis=0)
    last_g = _GSTEPS - 1
    args = [sc_matrix, seg, jnp.ones((D_MODEL, 1), _F32)]
    specs = [
        pl.BlockSpec((_SPS, N, N),
                     lambda b: (jnp.minimum(b, last_g), 0, 0)),
        pl.BlockSpec((D_MODEL, HEADS), lambda b: (0, 0)),
        pl.BlockSpec((D_MODEL, 1), lambda b: (0, 0)),
    ]
    for j in range(3):
        p = params['conv%d' % j]
        args += [p['W'], p['att_src'].reshape(1, D_MODEL),
                 p['att_dst'].reshape(1, D_MODEL)]
        specs += [pl.BlockSpec(p['W'].shape, lambda b: (0, 0)),
                  pl.BlockSpec((1, D_MODEL), lambda b: (0, 0)),
                  pl.BlockSpec((1, D_MODEL), lambda b: (0, 0))]
    args += [bnw, bnb]
    specs += [pl.BlockSpec((3, D_MODEL), lambda b: (0, 0)),
              pl.BlockSpec((3, D_MODEL), lambda b: (0, 0))]

    args += [params['W1'].reshape(N, D_MODEL, 256)]
    specs += [pl.BlockSpec(memory_space=pl.ANY)]

    args += [
        params['b1'].reshape(1, 256),
        params['ln1_w'].reshape(1, 256), params['ln1_b'].reshape(1, 256),
        params['W2'], params['b2'].reshape(1, 64),
        params['ln2_w'].reshape(1, 64), params['ln2_b'].reshape(1, 64),
        params['W3'], params['b3'].reshape(1, NUM_CLASSES),
    ]
    specs += [
        pl.BlockSpec((1, 256), lambda b: (0, 0)),
        pl.BlockSpec((1, 256), lambda b: (0, 0)),
        pl.BlockSpec((1, 256), lambda b: (0, 0)),
        pl.BlockSpec((256, 64), lambda b: (0, 0)),
        pl.BlockSpec((1, 64), lambda b: (0, 0)),
        pl.BlockSpec((1, 64), lambda b: (0, 0)),
        pl.BlockSpec((1, 64), lambda b: (0, 0)),
        pl.BlockSpec((64, NUM_CLASSES), lambda b: (0, 0)),
        pl.BlockSpec((1, NUM_CLASSES), lambda b: (0, 0)),
    ]

    out = pl.pallas_call(
        _fused_kernel,
        grid=(_STEPS,),
        in_specs=specs,
        out_specs=pl.BlockSpec((B, NUM_CLASSES), lambda b: (0, 0)),
        out_shape=jax.ShapeDtypeStruct((B, NUM_CLASSES), _F32),
        scratch_shapes=[
            pltpu.VMEM((B, N, D_MODEL), _F32),
            pltpu.VMEM((N, D_MODEL, 256), _F32),
            pltpu.SemaphoreType.DMA,
        ],
        compiler_params=pltpu.CompilerParams(
            dimension_semantics=(pltpu.ARBITRARY,),
            fuse_transposed_lhs_in_matmul=True),
    )(*args)
    return out
